# baseline (speedup 1.0000x reference)
import functools

import jax
import jax.numpy as jnp
from jax import lax
from jax.experimental import pallas as pl
from jax.experimental.pallas import tpu as pltpu
from jax.experimental.pallas import tpu_sc as plsc

DIM = 64
BBLK = 16


def _emb_body(idx_hbm, table_hbm, out_hbm,
              idx_v, rows_v, sem_i0, sem_i1, sem_o0, sem_o1, sem_g):
    sem_i = (sem_i0, sem_i1)
    sem_o = (sem_o0, sem_o1)
    wid = lax.axis_index("s") * 2 + lax.axis_index("c")
    B, L = idx_hbm.shape
    b_per_w = B // 32
    n_chunks = b_per_w // BBLK
    base_b = wid * b_per_w

    def idx_copy(g, b):
        return pltpu.async_copy(
            idx_hbm.at[pl.ds(base_b + g * BBLK, BBLK)],
            idx_v.at[b], sem_i[b])

    def out_copy(g, b):
        return pltpu.async_copy(
            rows_v.at[b],
            out_hbm.at[pl.ds(base_b + g * BBLK, BBLK)],
            sem_o[b])

    def wait_idx(b):
        pltpu.make_async_copy(
            idx_hbm.at[pl.ds(base_b, BBLK)], idx_v.at[b], sem_i[b]).wait()

    def wait_out(b):
        pltpu.make_async_copy(
            rows_v.at[b], out_hbm.at[pl.ds(base_b, BBLK)], sem_o[b]).wait()

    def gather(b):
        cps = [
            pltpu.async_copy(
                table_hbm.at[idx_v.at[b, j]],
                rows_v.at[b, j],
                sem_g)
            for j in range(BBLK)
        ]
        for cp in cps:
            cp.wait()

    idx_copy(0, 0)
    idx_copy(1, 1)
    for b in range(2):
        wait_idx(b)
        gather(b)
        out_copy(b, b)
        idx_copy(b + 2, b)

    def step(t, carry):
        for b in range(2):
            g = 2 * t + b
            wait_out(b)
            wait_idx(b)
            gather(b)
            out_copy(g, b)

            @pl.when(g + 2 < n_chunks)
            def _():
                idx_copy(g + 2, b)
        return carry

    lax.fori_loop(1, n_chunks // 2, step, 0)
    for b in range(2):
        wait_out(b)


def kernel(x, table):
    B, L = x.shape
    idx = x.astype(jnp.int32)

    mesh = plsc.VectorSubcoreMesh(core_axis_name="c", subcore_axis_name="s")
    emb = functools.partial(
        pl.kernel,
        mesh=mesh,
        out_type=jax.ShapeDtypeStruct((B, L, DIM), jnp.float32),
        scratch_types=[
            pltpu.VMEM((2, BBLK, L), jnp.int32),
            pltpu.VMEM((2, BBLK, L, DIM), jnp.float32),
            pltpu.SemaphoreType.DMA,
            pltpu.SemaphoreType.DMA,
            pltpu.SemaphoreType.DMA,
            pltpu.SemaphoreType.DMA,
            pltpu.SemaphoreType.DMA,
        ],
        compiler_params=pltpu.CompilerParams(use_tc_tiling_on_sc=False),
    )(_emb_body)

    return emb(idx, table)
